# baseline (speedup 1.0000x reference)
import functools

import jax
import jax.numpy as jnp
from jax.experimental import pallas as pl
from jax.experimental.pallas import tpu as pltpu
from jax.experimental.pallas import tpu_sc as plsc

D_MODEL = 64
SCALE = 8.0
LANES = 16
WINDOW = 128


def kernel(x, table):
    B = x.shape[0] * x.shape[1]
    idx = x.reshape(1, B).astype(jnp.int32)
    mesh = plsc.VectorSubcoreMesh(core_axis_name="core", subcore_axis_name="subcore")

    @functools.partial(
        pl.kernel,
        out_type=jax.ShapeDtypeStruct((B, D_MODEL), table.dtype),
        mesh=mesh,
        compiler_params=pltpu.CompilerParams(use_tc_tiling_on_sc=False),
    )
    def gather_scale(table_hbm, idx_hbm, out_hbm):
        def body(i_vmem, o_vmem):
            pltpu.sync_copy(table_hbm.at[i_vmem.at[0]], o_vmem)

            @pl.loop(0, WINDOW)
            def _(r):
                for c in range(D_MODEL // LANES):
                    sl = (r, pl.ds(c * LANES, LANES))
                    o_vmem[sl] = o_vmem[sl] * SCALE

        pltpu.emit_pipeline(
            body,
            grid=(B // WINDOW,),
            in_specs=[pl.BlockSpec((1, WINDOW), index_map=lambda i: (0, i))],
            out_specs=[pl.BlockSpec((WINDOW, D_MODEL), index_map=lambda i: (i, 0))],
            core_axis_name=("core", "subcore"),
            dimension_semantics=(pltpu.PARALLEL,),
        )(idx_hbm, out_hbm)

    out = gather_scale(table, idx)
    return out.reshape(x.shape[0], x.shape[1], D_MODEL)


# speedup vs baseline: 1.4966x; 1.4966x over previous
import functools

import jax
from jax import lax
import jax.numpy as jnp
from jax.experimental import pallas as pl
from jax.experimental.pallas import tpu as pltpu
from jax.experimental.pallas import tpu_sc as plsc

D_MODEL = 64
SCALE = 8.0
LANES = 16
C = 128
NBUF = 4


def kernel(x, table):
    B = x.shape[0] * x.shape[1]
    NW = 32
    per_w = B // NW
    n_chunks = per_w // C
    n_groups = n_chunks // NBUF

    idx2d = x.reshape(B // C, C).astype(jnp.int32)
    mesh = plsc.VectorSubcoreMesh(core_axis_name="core", subcore_axis_name="subcore")

    @functools.partial(
        pl.kernel,
        out_type=jax.ShapeDtypeStruct((B, D_MODEL), table.dtype),
        mesh=mesh,
        scratch_types=[
            pltpu.VMEM((n_chunks, C), jnp.int32),
            pltpu.VMEM((NBUF, C, D_MODEL), jnp.float32),
            pltpu.VMEM((NBUF, C, D_MODEL), jnp.float32),
            pltpu.SemaphoreType.DMA,
            [pltpu.SemaphoreType.DMA] * NBUF,
            [pltpu.SemaphoreType.DMA] * NBUF,
        ],
        compiler_params=pltpu.CompilerParams(use_tc_tiling_on_sc=False),
    )
    def gather_scale(table_hbm, idx_hbm, out_hbm, idx_v, in_v, out_v,
                     sem_i, sem_g, sem_o):
        wid = lax.axis_index("subcore") * 2 + lax.axis_index("core")
        chunk0 = wid * n_chunks
        pltpu.async_copy(idx_hbm.at[pl.ds(chunk0, n_chunks)], idx_v, sem_i).wait()

        def start_gather(step, b):
            pltpu.make_async_copy(
                table_hbm.at[idx_v.at[step]], in_v.at[b], sem_g[b]).start()

        def wait_gather(step, b):
            pltpu.make_async_copy(
                table_hbm.at[idx_v.at[step]], in_v.at[b], sem_g[b]).wait()

        def start_out(step, b):
            pltpu.make_async_copy(
                out_v.at[b], out_hbm.at[pl.ds((chunk0 + step) * C, C)],
                sem_o[b]).start()

        def wait_out(step, b):
            pltpu.make_async_copy(
                out_v.at[b], out_hbm.at[pl.ds((chunk0 + step) * C, C)],
                sem_o[b]).wait()

        for b in range(NBUF):
            start_gather(b, b)

        @pl.loop(0, n_groups)
        def _(g):
            step0 = g * NBUF
            for b in range(NBUF):
                wait_gather(step0 + b, b)

                @pl.when(g > 0)
                def _():
                    wait_out(step0 + b - NBUF, b)

                @pl.loop(0, C, step=8)
                def _(r):
                    for k in range(8):
                        for c in range(D_MODEL // LANES):
                            sl = (r + k, pl.ds(c * LANES, LANES))
                            out_v[(b,) + sl] = in_v[(b,) + sl] * SCALE

                start_out(step0 + b, b)

                @pl.when(g < n_groups - 1)
                def _():
                    start_gather(step0 + b + NBUF, b)

        for b in range(NBUF):
            wait_out(n_chunks - NBUF + b, b)

    out = gather_scale(table, idx2d)
    return out.reshape(x.shape[0], x.shape[1], D_MODEL)
